# baseline (speedup 1.0000x reference)
import functools

import jax
import jax.numpy as jnp
from jax import lax
from jax.experimental import pallas as pl
from jax.experimental.pallas import tpu as pltpu
from jax.experimental.pallas import tpu_sc as plsc

X_SIZE = 32
HIDDEN = 128
EPS = 1e-12

NC = 2
NS = 16
NW = NC * NS
LPAD = 56
NB = 112
NPIECES = 4
BB = 128


def _gather_sum_body(piece_base, tok_per_w, wl_ids, pos_ids, hop_ids,
                     wl_t, pos_t, hop_t, out, iw, ip, ih,
                     bw0, bp0, bh0, bw1, bp1, bh1, gs0, gs1, ss0, ss1):
    sid = lax.axis_index("s")
    wid = sid * NC + lax.axis_index("c")
    rel_base = wid * tok_per_w
    base = piece_base + rel_base
    num_chunks = tok_per_w // NB

    pltpu.sync_copy(wl_ids.at[pl.ds(base, tok_per_w)], iw)
    pltpu.sync_copy(pos_ids.at[pl.ds(base, tok_per_w)], ip)
    pltpu.sync_copy(hop_ids.at[pl.ds(base, tok_per_w)], ih)

    def issue(g, bw, bp, bh, sem):
        pltpu.async_copy(wl_t.at[iw.at[pl.ds(g * NB, NB)]], bw, sem)
        pltpu.async_copy(pos_t.at[ip.at[pl.ds(g * NB, NB)]], bp, sem)
        pltpu.async_copy(hop_t.at[ih.at[pl.ds(g * NB, NB)]], bh, sem)

    def drain_gathers(g, bw, bp, bh, sem):
        pltpu.make_async_copy(wl_t.at[iw.at[pl.ds(g * NB, NB)]], bw, sem).wait()
        pltpu.make_async_copy(pos_t.at[ip.at[pl.ds(g * NB, NB)]], bp, sem).wait()
        pltpu.make_async_copy(hop_t.at[ih.at[pl.ds(g * NB, NB)]], bh, sem).wait()

    def add_and_store(g, bw, bp, bh, sem):
        def row(t, c2):
            for cc in range(HIDDEN // 16):
                sl = pl.ds(cc * 16, 16)
                plsc.addupdate(bw.at[t, sl], bp[t, sl] + bh[t, sl])
            return c2

        lax.fori_loop(0, NB, row, 0)
        pltpu.async_copy(bw, out.at[pl.ds(rel_base + g * NB, NB)], sem)

    def drain_scatter(bw, sem):
        pltpu.make_async_copy(bw, out.at[pl.ds(0, NB)], sem).wait()

    issue(0, bw0, bp0, bh0, gs0)

    def pair(k, carry):
        g0 = 2 * k
        g1 = g0 + 1

        @pl.when(k > 0)
        def _():
            drain_scatter(bw1, ss1)

        issue(g1, bw1, bp1, bh1, gs1)
        drain_gathers(g0, bw0, bp0, bh0, gs0)
        add_and_store(g0, bw0, bp0, bh0, ss0)

        @pl.when(k < num_chunks // 2 - 1)
        def _():
            drain_scatter(bw0, ss0)
            issue(g0 + 2, bw0, bp0, bh0, gs0)

        drain_gathers(g1, bw1, bp1, bh1, gs1)
        add_and_store(g1, bw1, bp1, bh1, ss1)
        return carry

    lax.fori_loop(0, num_chunks // 2, pair, 0)
    drain_scatter(bw0, ss0)
    drain_scatter(bw1, ss1)


def _proj_ln_compute(raw_ref, gsum_ref, w_ref, b_ref, g_ref, be_ref, out_ref):
    proj = lax.dot_general(raw_ref[...], w_ref[...],
                           (((2,), (0,)), ((), ())),
                           preferred_element_type=jnp.float32)
    e = proj + b_ref[...] + gsum_ref[:, :50, :]
    mean = jnp.mean(e, axis=2, keepdims=True)
    cent = e - mean
    var = jnp.mean(cent * cent, axis=2, keepdims=True)
    normed = cent * lax.rsqrt(var + EPS)
    out_ref[...] = normed * g_ref[...] + be_ref[...]


def _proj_ln_body(raw_ref, gsum_ref, w_ref, b_ref, g_ref, be_ref, out_ref):
    _proj_ln_compute(raw_ref, gsum_ref, w_ref, b_ref, g_ref, be_ref, out_ref)


def _proj_ln_body_acc(raw_ref, gsum_ref, w_ref, b_ref, g_ref, be_ref,
                      carry_ref, out_ref):
    del carry_ref
    _proj_ln_compute(raw_ref, gsum_ref, w_ref, b_ref, g_ref, be_ref, out_ref)


def kernel(raw_features, wl_role_ids, init_pos_ids, hop_dis_ids, W, b,
           wl_table, pos_table, hop_table, gamma, beta):
    Bb, Ll, X = raw_features.shape
    Np = Bb * LPAD
    npc = Np // NPIECES
    tok_per_w = npc // NW
    rows_pp = Bb // NPIECES
    bpp = rows_pp // BB

    pad = ((0, 0), (0, LPAD - Ll))
    wl_ids = jnp.pad(wl_role_ids, pad).reshape(Np).astype(jnp.int32)
    pos_ids = jnp.pad(init_pos_ids, pad).reshape(Np).astype(jnp.int32)
    hop_ids = jnp.pad(hop_dis_ids, pad).reshape(Np).astype(jnp.int32)
    b2 = b.reshape(1, 1, HIDDEN)
    g2 = gamma.reshape(1, 1, HIDDEN)
    be2 = beta.reshape(1, 1, HIDDEN)

    sc_scratch = [
        pltpu.VMEM((tok_per_w,), jnp.int32),
        pltpu.VMEM((tok_per_w,), jnp.int32),
        pltpu.VMEM((tok_per_w,), jnp.int32),
        pltpu.VMEM((NB, HIDDEN), jnp.float32),
        pltpu.VMEM((NB, HIDDEN), jnp.float32),
        pltpu.VMEM((NB, HIDDEN), jnp.float32),
        pltpu.VMEM((NB, HIDDEN), jnp.float32),
        pltpu.VMEM((NB, HIDDEN), jnp.float32),
        pltpu.VMEM((NB, HIDDEN), jnp.float32),
        pltpu.SemaphoreType.DMA,
        pltpu.SemaphoreType.DMA,
        pltpu.SemaphoreType.DMA,
        pltpu.SemaphoreType.DMA,
    ]
    mesh = plsc.VectorSubcoreMesh(core_axis_name="c", subcore_axis_name="s")

    def gather_piece(h):
        fn = pl.kernel(
            functools.partial(_gather_sum_body, h * npc, tok_per_w),
            out_type=jax.ShapeDtypeStruct((npc, HIDDEN), jnp.float32),
            mesh=mesh,
            scratch_types=sc_scratch,
        )
        return fn(wl_ids, pos_ids, hop_ids, wl_table, pos_table, hop_table)

    def ln_piece(h, gsum_h, carry):
        gsum3 = gsum_h.reshape(rows_pp, LPAD, HIDDEN)
        raw_spec = pl.BlockSpec((BB, Ll, X), lambda i, h=h: (h * bpp + i, 0, 0))
        out_spec = pl.BlockSpec((BB, Ll, HIDDEN),
                                lambda i, h=h: (h * bpp + i, 0, 0))
        gs_spec = pl.BlockSpec((BB, LPAD, HIDDEN), lambda i: (i, 0, 0))
        full = lambda i: (0, 0, 0)
        in_specs = [
            raw_spec,
            gs_spec,
            pl.BlockSpec((X_SIZE, HIDDEN), lambda i: (0, 0)),
            pl.BlockSpec((1, 1, HIDDEN), full),
            pl.BlockSpec((1, 1, HIDDEN), full),
            pl.BlockSpec((1, 1, HIDDEN), full),
        ]
        args = [raw_features, gsum3, W, b2, g2, be2]
        body = _proj_ln_body
        io_aliases = {}
        if carry is not None:
            in_specs.append(pl.BlockSpec(memory_space=pl.ANY))
            args.append(carry)
            body = _proj_ln_body_acc
            io_aliases = {6: 0}
        return pl.pallas_call(
            body,
            grid=(bpp,),
            in_specs=in_specs,
            out_specs=out_spec,
            out_shape=jax.ShapeDtypeStruct((Bb, Ll, HIDDEN), jnp.float32),
            input_output_aliases=io_aliases,
        )(*args)

    out = None
    for h in range(NPIECES):
        gsum_h = gather_piece(h)
        out = ln_piece(h, gsum_h, out)

    return out


# speedup vs baseline: 3.6767x; 3.6767x over previous
import functools

import jax
import jax.numpy as jnp
from jax import lax
from jax.experimental import pallas as pl
from jax.experimental.pallas import tpu as pltpu
from jax.experimental.pallas import tpu_sc as plsc

X_SIZE = 32
HIDDEN = 128
EPS = 1e-12

NC = 2
NS = 16
NW = NC * NS
NB = 80
NPIECES = 4
BB = 128


def _gather_sum_body(piece_base, tok_per_w, wl_ids, pos_ids, hop_ids,
                     wl_t, pos_t, hop_t, out, iw, ip, ih,
                     bw0, bp0, bh0, bw1, bp1, bh1, gs0, gs1, ss0, ss1):
    sid = lax.axis_index("s")
    wid = sid * NC + lax.axis_index("c")
    rel_base = wid * tok_per_w
    base = piece_base + rel_base
    num_chunks = tok_per_w // NB

    pltpu.sync_copy(wl_ids.at[pl.ds(base, tok_per_w)], iw)
    pltpu.sync_copy(pos_ids.at[pl.ds(base, tok_per_w)], ip)
    pltpu.sync_copy(hop_ids.at[pl.ds(base, tok_per_w)], ih)

    def issue(g, bw, bp, bh, sem):
        pltpu.async_copy(wl_t.at[iw.at[pl.ds(g * NB, NB)]], bw, sem)
        pltpu.async_copy(pos_t.at[ip.at[pl.ds(g * NB, NB)]], bp, sem)
        pltpu.async_copy(hop_t.at[ih.at[pl.ds(g * NB, NB)]], bh, sem)

    def drain_gathers(g, bw, bp, bh, sem):
        pltpu.make_async_copy(wl_t.at[iw.at[pl.ds(g * NB, NB)]], bw, sem).wait()
        pltpu.make_async_copy(pos_t.at[ip.at[pl.ds(g * NB, NB)]], bp, sem).wait()
        pltpu.make_async_copy(hop_t.at[ih.at[pl.ds(g * NB, NB)]], bh, sem).wait()

    def add_and_store(g, bw, bp, bh, sem):
        def row(t, c2):
            for cc in range(HIDDEN // 16):
                sl = pl.ds(cc * 16, 16)
                plsc.addupdate(bw.at[t, sl], bp[t, sl] + bh[t, sl])
            return c2

        lax.fori_loop(0, NB, row, 0)
        pltpu.async_copy(bw, out.at[pl.ds(rel_base + g * NB, NB)], sem)

    def drain_scatter(bw, sem):
        pltpu.make_async_copy(bw, out.at[pl.ds(0, NB)], sem).wait()

    issue(0, bw0, bp0, bh0, gs0)

    def pair(k, carry):
        g0 = 2 * k
        g1 = g0 + 1

        @pl.when(k > 0)
        def _():
            drain_scatter(bw1, ss1)

        issue(g1, bw1, bp1, bh1, gs1)
        drain_gathers(g0, bw0, bp0, bh0, gs0)
        add_and_store(g0, bw0, bp0, bh0, ss0)

        @pl.when(k < num_chunks // 2 - 1)
        def _():
            drain_scatter(bw0, ss0)
            issue(g0 + 2, bw0, bp0, bh0, gs0)

        drain_gathers(g1, bw1, bp1, bh1, gs1)
        add_and_store(g1, bw1, bp1, bh1, ss1)
        return carry

    lax.fori_loop(0, num_chunks // 2, pair, 0)
    drain_scatter(bw0, ss0)
    drain_scatter(bw1, ss1)


def _proj_ln_compute(raw_ref, gsum_ref, w_ref, b_ref, g_ref, be_ref, out_ref):
    bb, ll, x = raw_ref.shape
    x2 = raw_ref[...].reshape(bb * ll, x)
    proj = jnp.dot(x2, w_ref[...], preferred_element_type=jnp.float32)
    e = proj + b_ref[...] + gsum_ref[...]
    mean = jnp.mean(e, axis=1, keepdims=True)
    cent = e - mean
    var = jnp.mean(cent * cent, axis=1, keepdims=True)
    normed = cent * lax.rsqrt(var + EPS)
    res = normed * g_ref[...] + be_ref[...]
    out_ref[...] = res.reshape(bb, ll, HIDDEN)


def _proj_ln_body(raw_ref, gsum_ref, w_ref, b_ref, g_ref, be_ref, out_ref):
    _proj_ln_compute(raw_ref, gsum_ref, w_ref, b_ref, g_ref, be_ref, out_ref)


def _proj_ln_body_acc(raw_ref, gsum_ref, w_ref, b_ref, g_ref, be_ref,
                      carry_ref, out_ref):
    del carry_ref
    _proj_ln_compute(raw_ref, gsum_ref, w_ref, b_ref, g_ref, be_ref, out_ref)


def kernel(raw_features, wl_role_ids, init_pos_ids, hop_dis_ids, W, b,
           wl_table, pos_table, hop_table, gamma, beta):
    Bb, Ll, X = raw_features.shape
    N = Bb * Ll
    npc = N // NPIECES
    tok_per_w = npc // NW
    rows_pp = Bb // NPIECES
    bpp = rows_pp // BB

    wl_ids = wl_role_ids.reshape(N).astype(jnp.int32)
    pos_ids = init_pos_ids.reshape(N).astype(jnp.int32)
    hop_ids = hop_dis_ids.reshape(N).astype(jnp.int32)
    b2 = b.reshape(1, HIDDEN)
    g2 = gamma.reshape(1, HIDDEN)
    be2 = beta.reshape(1, HIDDEN)

    sc_scratch = [
        pltpu.VMEM((tok_per_w,), jnp.int32),
        pltpu.VMEM((tok_per_w,), jnp.int32),
        pltpu.VMEM((tok_per_w,), jnp.int32),
        pltpu.VMEM((NB, HIDDEN), jnp.float32),
        pltpu.VMEM((NB, HIDDEN), jnp.float32),
        pltpu.VMEM((NB, HIDDEN), jnp.float32),
        pltpu.VMEM((NB, HIDDEN), jnp.float32),
        pltpu.VMEM((NB, HIDDEN), jnp.float32),
        pltpu.VMEM((NB, HIDDEN), jnp.float32),
        pltpu.SemaphoreType.DMA,
        pltpu.SemaphoreType.DMA,
        pltpu.SemaphoreType.DMA,
        pltpu.SemaphoreType.DMA,
    ]
    mesh = plsc.VectorSubcoreMesh(core_axis_name="c", subcore_axis_name="s")

    def gather_piece(h):
        fn = pl.kernel(
            functools.partial(_gather_sum_body, h * npc, tok_per_w),
            out_type=jax.ShapeDtypeStruct((npc, HIDDEN), jnp.float32),
            mesh=mesh,
            scratch_types=sc_scratch,
        )
        return fn(wl_ids, pos_ids, hop_ids, wl_table, pos_table, hop_table)

    def ln_piece(h, gsum_h, carry):
        raw_spec = pl.BlockSpec((BB, Ll, X), lambda i, h=h: (h * bpp + i, 0, 0))
        out_spec = pl.BlockSpec((BB, Ll, HIDDEN),
                                lambda i, h=h: (h * bpp + i, 0, 0))
        gs_spec = pl.BlockSpec((BB * Ll, HIDDEN), lambda i: (i, 0))
        full = lambda i: (0, 0)
        in_specs = [
            raw_spec,
            gs_spec,
            pl.BlockSpec((X_SIZE, HIDDEN), full),
            pl.BlockSpec((1, HIDDEN), full),
            pl.BlockSpec((1, HIDDEN), full),
            pl.BlockSpec((1, HIDDEN), full),
        ]
        args = [raw_features, gsum_h, W, b2, g2, be2]
        body = _proj_ln_body
        io_aliases = {}
        if carry is not None:
            in_specs.append(pl.BlockSpec(memory_space=pl.ANY))
            args.append(carry)
            body = _proj_ln_body_acc
            io_aliases = {6: 0}
        return pl.pallas_call(
            body,
            grid=(bpp,),
            in_specs=in_specs,
            out_specs=out_spec,
            out_shape=jax.ShapeDtypeStruct((Bb, Ll, HIDDEN), jnp.float32),
            input_output_aliases=io_aliases,
        )(*args)

    out = None
    for h in range(NPIECES):
        gsum_h = gather_piece(h)
        out = ln_piece(h, gsum_h, out)

    return out
